# baseline (speedup 1.0000x reference)
import functools

import jax
import jax.numpy as jnp
from jax import lax
from jax.experimental import pallas as pl
from jax.experimental.pallas import tpu as pltpu
from jax.experimental.pallas import tpu_sc as plsc

N_MASKS = 100000
N_PROTOTYPES = 64
BATCH = 16384

_NC = 2
_NS = 16
_NW = _NC * _NS
_LANES = 16

_FPW = N_PROTOTYPES // _NW
_NQ = 4
_QB = BATCH // _NQ
_NB = _QB // _LANES

_mesh = plsc.VectorSubcoreMesh(core_axis_name="c", subcore_axis_name="s")


@functools.partial(
    pl.kernel,
    out_type=jax.ShapeDtypeStruct((N_PROTOTYPES, BATCH), jnp.float32),
    mesh=_mesh,
    scratch_types=[
        pltpu.VMEM((N_MASKS,), jnp.float32),
        pltpu.VMEM((BATCH,), jnp.int32),
        pltpu.VMEM_SHARED((BATCH,), jnp.int32),
        pltpu.VMEM((_QB,), jnp.float32),
        pltpu.VMEM((_QB,), jnp.float32),
        pltpu.SemaphoreType.DMA,
        pltpu.SemaphoreType.DMA,
        pltpu.SemaphoreType.DMA,
        pltpu.SemaphoreType.DMA,
        pltpu.SemaphoreType.DMA,
    ],
    compiler_params=pltpu.CompilerParams(
        use_tc_tiling_on_sc=True, needs_layout_passes=False
    ),
)
def _proto_mul(
    table_hbm, idx_hbm, in_hbm, out_hbm,
    plane_v, idx_v, idx_sh, io0, io1, sem_p, sem_i, sem_l, sem_s0, sem_s1,
):
    sid = lax.axis_index("s")
    wid = sid * _NC + lax.axis_index("c")
    bufs = (io0, io1)
    ssems = (sem_s0, sem_s1)

    plane_cp = pltpu.async_copy(table_hbm.at[wid], plane_v, sem_p)
    first_load = pltpu.async_copy(in_hbm.at[wid, pl.ds(0, _QB)], bufs[0], sem_l)

    seg = BATCH // _NS
    pltpu.async_copy(
        idx_hbm.at[pl.ds(sid * seg, seg)], idx_sh.at[pl.ds(sid * seg, seg)], sem_i
    ).wait()
    plsc.subcore_barrier()
    idx_cp = pltpu.async_copy(idx_sh, idx_v, sem_i)

    for fp in range(_FPW):
        f = wid + fp * _NW
        if fp > 0:
            plane_cp = pltpu.async_copy(table_hbm.at[f], plane_v, sem_p)
            first_load = pltpu.async_copy(
                in_hbm.at[f, pl.ds(0, _QB)], bufs[0], sem_l
            )
        loads = [first_load]
        stores = [None, None]
        plane_cp.wait()
        if fp == 0:
            idx_cp.wait()

        for q in range(_NQ):
            cur = bufs[q % 2]
            if q + 1 < _NQ:
                nxt = bufs[(q + 1) % 2]
                if stores[(q + 1) % 2] is not None:
                    stores[(q + 1) % 2].wait()
                    stores[(q + 1) % 2] = None
                loads.append(
                    pltpu.async_copy(
                        in_hbm.at[f, pl.ds((q + 1) * _QB, _QB)], nxt, sem_l
                    )
                )
            loads[q].wait()

            qbase = q * _QB

            @plsc.parallel_loop(0, _NB, unroll=4)
            def _blk(bb):
                sl = pl.ds(bb * _LANES, _LANES)
                iv = idx_v[pl.ds(qbase + bb * _LANES, _LANES)]
                x = plsc.load_gather(plane_v, [iv])
                y = cur[sl]
                cur[sl] = y / (1.0 + jnp.exp(-x))

            stores[q % 2] = pltpu.async_copy(
                cur, out_hbm.at[f, pl.ds(qbase, _QB)], ssems[q % 2]
            )
        for s in stores:
            if s is not None:
                s.wait()


def kernel(in_repr, mask_idx, prototype_knobs):
    out_t = _proto_mul(prototype_knobs.T, mask_idx.astype(jnp.int32), in_repr.T)
    return out_t.T


# speedup vs baseline: 1.0033x; 1.0033x over previous
import functools

import jax
import jax.numpy as jnp
from jax import lax
from jax.experimental import pallas as pl
from jax.experimental.pallas import tpu as pltpu
from jax.experimental.pallas import tpu_sc as plsc

N_MASKS = 100000
N_PROTOTYPES = 64
BATCH = 16384

_NC = 2
_NS = 16
_NW = _NC * _NS
_LANES = 16

_FPW = N_PROTOTYPES // _NW
_NQ = 4
_QB = BATCH // _NQ
_NB = _QB // _LANES

_mesh = plsc.VectorSubcoreMesh(core_axis_name="c", subcore_axis_name="s")


@functools.partial(
    pl.kernel,
    out_type=jax.ShapeDtypeStruct((N_PROTOTYPES, BATCH), jnp.float32),
    mesh=_mesh,
    scratch_types=[
        pltpu.VMEM((N_MASKS,), jnp.float32),
        pltpu.VMEM((BATCH,), jnp.int32),
        pltpu.VMEM_SHARED((BATCH,), jnp.int32),
        pltpu.VMEM((_QB,), jnp.float32),
        pltpu.VMEM((_QB,), jnp.float32),
        pltpu.SemaphoreType.DMA,
        pltpu.SemaphoreType.DMA,
        pltpu.SemaphoreType.DMA,
        pltpu.SemaphoreType.DMA,
        pltpu.SemaphoreType.DMA,
    ],
    compiler_params=pltpu.CompilerParams(
        use_tc_tiling_on_sc=True, needs_layout_passes=False
    ),
)
def _proto_mul(
    table_hbm, idx_hbm, in_hbm, out_hbm,
    plane_v, idx_v, idx_sh, io0, io1, sem_p, sem_i, sem_l, sem_s0, sem_s1,
):
    sid = lax.axis_index("s")
    wid = sid * _NC + lax.axis_index("c")
    bufs = (io0, io1)
    ssems = (sem_s0, sem_s1)

    plane_cp = pltpu.async_copy(table_hbm.at[wid], plane_v, sem_p)
    first_load = pltpu.async_copy(in_hbm.at[wid, pl.ds(0, _QB)], bufs[0], sem_l)

    seg = BATCH // _NS
    pltpu.async_copy(
        idx_hbm.at[pl.ds(sid * seg, seg)], idx_sh.at[pl.ds(sid * seg, seg)], sem_i
    ).wait()
    plsc.subcore_barrier()
    idx_cp = pltpu.async_copy(idx_sh, idx_v, sem_i)

    for fp in range(_FPW):
        f = wid + fp * _NW
        if fp > 0:
            plane_cp = pltpu.async_copy(table_hbm.at[f], plane_v, sem_p)
            first_load = pltpu.async_copy(
                in_hbm.at[f, pl.ds(0, _QB)], bufs[0], sem_l
            )
        loads = [first_load]
        stores = [None, None]
        plane_cp.wait()
        if fp == 0:
            idx_cp.wait()

        for q in range(_NQ):
            cur = bufs[q % 2]
            if q + 1 < _NQ:
                nxt = bufs[(q + 1) % 2]
                if stores[(q + 1) % 2] is not None:
                    stores[(q + 1) % 2].wait()
                    stores[(q + 1) % 2] = None
                loads.append(
                    pltpu.async_copy(
                        in_hbm.at[f, pl.ds((q + 1) * _QB, _QB)], nxt, sem_l
                    )
                )
            loads[q].wait()

            qbase = q * _QB

            @plsc.parallel_loop(0, _NB, unroll=8)
            def _blk(bb):
                sl = pl.ds(bb * _LANES, _LANES)
                iv = idx_v[pl.ds(qbase + bb * _LANES, _LANES)]
                x = plsc.load_gather(plane_v, [iv])
                y = cur[sl]
                cur[sl] = y / (1.0 + jnp.exp(-x))

            stores[q % 2] = pltpu.async_copy(
                cur, out_hbm.at[f, pl.ds(qbase, _QB)], ssems[q % 2]
            )
        for s in stores:
            if s is not None:
                s.wait()


def kernel(in_repr, mask_idx, prototype_knobs):
    out_t = _proto_mul(prototype_knobs.T, mask_idx.astype(jnp.int32), in_repr.T)
    return out_t.T
